# baseline (speedup 1.0000x reference)
import functools

import jax
import jax.numpy as jnp
from jax import lax
from jax.experimental import pallas as pl
from jax.experimental.pallas import tpu as pltpu
from jax.experimental.pallas import tpu_sc as plsc

N = 100000
D = 128
G = 512
NC = 2
NS = 16
NW = NC * NS
CH = 128
NCHUNKS = -(-N // CH)
TAIL = N - (NCHUNKS - 1) * CH
MAXCH = -(-NCHUNKS // NW)
GPS = G // NS
GCH = 2 * CH
NBUF = 3
LASTW = (NCHUNKS - 1) // MAXCH


def _sc_body(emb, idxh, zeros, out, rows_v, idx_v, acc, gsem, ssem):
    c = lax.axis_index("c")
    s = lax.axis_index("s")
    w = c * NS + s
    start = w * MAXCH
    nch = jnp.clip(NCHUNKS - start, 0, MAXCH)
    nfull = nch - jnp.where(w == LASTW, 1, 0)
    ngf = nfull // 2
    ng = ngf + lax.rem(nfull, 2)

    pltpu.sync_copy(zeros, rows_v.at[0, pl.ds(0, CH)])
    pltpu.sync_copy(rows_v.at[0, pl.ds(0, GPS)], acc.at[pl.ds(s * GPS, GPS)])
    pltpu.sync_copy(idxh.at[w], idx_v)

    @pl.when(w == LASTW)
    def _():
        pltpu.sync_copy(zeros, rows_v.at[NBUF - 1, pl.ds(0, CH)])

    def gather(j):
        b = lax.rem(j, NBUF)

        @pl.when(j < ngf)
        def _():
            pltpu.async_copy(emb.at[pl.ds((start + 2 * j) * CH, GCH)],
                             rows_v.at[b], gsem.at[b])

        @pl.when(j >= ngf)
        def _():
            pltpu.async_copy(emb.at[pl.ds((start + 2 * j) * CH, CH)],
                             rows_v.at[b, pl.ds(0, CH)], gsem.at[b])

    def wait_gather(j, b):
        @pl.when(j < ngf)
        def _():
            pltpu.make_async_copy(emb.at[pl.ds(0, GCH)], rows_v.at[b],
                                  gsem.at[b]).wait()

        @pl.when(j >= ngf)
        def _():
            pltpu.make_async_copy(emb.at[pl.ds(0, CH)],
                                  rows_v.at[b, pl.ds(0, CH)],
                                  gsem.at[b]).wait()

    def wait_scatter_iter(i):
        b = lax.rem(i, NBUF)
        pltpu.make_async_copy(rows_v.at[b, pl.ds(0, CH)], acc.at[idx_v.at[0]],
                              ssem.at[b]).wait()

        @pl.when(2 * i + 1 < nfull)
        def _():
            pltpu.make_async_copy(rows_v.at[b, pl.ds(0, CH)],
                                  acc.at[idx_v.at[0]], ssem.at[b]).wait()

    for j0 in range(NBUF - 1):
        @pl.when(j0 < ng)
        def _():
            gather(j0)

    plsc.subcore_barrier()

    @pl.when(w == LASTW)
    def _():
        rb = (NCHUNKS - 1) * CH
        pltpu.sync_copy(emb.at[pl.ds(rb, TAIL)],
                        rows_v.at[NBUF - 1, pl.ds(0, TAIL)])
        pltpu.sync_copy(rows_v.at[NBUF - 1, pl.ds(0, CH)],
                        acc.at[idx_v.at[nch - 1]], add=True)

    def step(j, carry):
        b = lax.rem(j, NBUF)

        @pl.when(j + (NBUF - 1) < ng)
        def _():
            @pl.when(j >= 1)
            def _():
                wait_scatter_iter(j - 1)
            gather(j + (NBUF - 1))

        wait_gather(j, b)
        pltpu.async_copy(rows_v.at[b, pl.ds(0, CH)], acc.at[idx_v.at[2 * j]],
                         ssem.at[b], add=True)

        @pl.when(2 * j + 1 < nfull)
        def _():
            pltpu.async_copy(rows_v.at[b, pl.ds(CH, CH)],
                             acc.at[idx_v.at[2 * j + 1]], ssem.at[b],
                             add=True)
        return carry

    lax.fori_loop(0, ng, step, 0)

    def drain(i, carry):
        wait_scatter_iter(i)
        return carry

    lax.fori_loop(jnp.maximum(ng - NBUF, 0), ng, drain, 0)
    plsc.subcore_barrier()
    pltpu.sync_copy(acc.at[pl.ds(s * GPS, GPS)], out.at[c, pl.ds(s * GPS, GPS)])


_sc_segsum = functools.partial(
    pl.kernel,
    out_type=jax.ShapeDtypeStruct((NC, G, D), jnp.float32),
    mesh=plsc.VectorSubcoreMesh(core_axis_name="c", subcore_axis_name="s"),
    name="sc_segment_sum",
    scratch_types=[
        pltpu.VMEM((NBUF, GCH, D), jnp.float32),
        pltpu.VMEM((MAXCH, CH), jnp.int32),
        pltpu.VMEM_SHARED((G, D), jnp.float32),
        pltpu.SemaphoreType.DMA((NBUF,)),
        pltpu.SemaphoreType.DMA((NBUF,)),
    ],
)(_sc_body)


def _mlp_body(p_ref, w1_ref, b1_ref, w2_ref, b2_ref, o_ref):
    g = p_ref[0] + p_ref[1]
    h = lax.dot_general(g, w1_ref[...], (((1,), (1,)), ((), ())),
                        preferred_element_type=jnp.float32)
    h = jnp.maximum(h + b1_ref[...], 0.0)
    o_ref[...] = lax.dot_general(h, w2_ref[...], (((1,), (1,)), ((), ())),
                                 preferred_element_type=jnp.float32) + b2_ref[...]


_tc_mlp = pl.pallas_call(
    _mlp_body,
    out_shape=jax.ShapeDtypeStruct((G, D), jnp.float32),
)


def kernel(node_embeddings, batch_indices, W1, b1, W2, b2):
    idx = batch_indices.astype(jnp.int32)
    idx3 = jnp.pad(idx, (0, NW * MAXCH * CH - N)).reshape(NW, MAXCH, CH)
    zeros = jnp.zeros((CH, D), jnp.float32)
    partials = _sc_segsum(node_embeddings, idx3, zeros)
    return _tc_mlp(partials, W1, b1.reshape(1, D), W2, b2.reshape(1, D))


# speedup vs baseline: 1.0303x; 1.0303x over previous
import functools

import jax
import jax.numpy as jnp
from jax import lax
from jax.experimental import pallas as pl
from jax.experimental.pallas import tpu as pltpu
from jax.experimental.pallas import tpu_sc as plsc

N = 100000
D = 128
G = 512
NC = 2
NS = 16
NW = NC * NS
CH = 128
NCHUNKS = -(-N // CH)
TAIL = N - (NCHUNKS - 1) * CH
MAXCH = -(-NCHUNKS // NW)
GPS = G // NS
NBUF = 7
LASTW = (NCHUNKS - 1) // MAXCH


def _sc_body(emb, idxh, zeros, out, rows_v, idx_v, acc, gsem, ssem):
    c = lax.axis_index("c")
    s = lax.axis_index("s")
    w = c * NS + s
    start = w * MAXCH
    nch = jnp.clip(NCHUNKS - start, 0, MAXCH)
    nfull = nch - jnp.where(w == LASTW, 1, 0)

    pltpu.sync_copy(zeros, rows_v.at[0])
    pltpu.sync_copy(rows_v.at[0, pl.ds(0, GPS)], acc.at[pl.ds(s * GPS, GPS)])
    pltpu.sync_copy(idxh.at[w], idx_v)
    plsc.subcore_barrier()

    @pl.when(w == LASTW)
    def _():
        rb = (NCHUNKS - 1) * CH
        pltpu.sync_copy(emb.at[pl.ds(rb, TAIL)], rows_v.at[0, pl.ds(0, TAIL)])
        pltpu.sync_copy(rows_v.at[0], acc.at[idx_v.at[nch - 1]], add=True)

    def gather(k):
        b = lax.rem(k, NBUF)
        pltpu.async_copy(emb.at[pl.ds((start + k) * CH, CH)], rows_v.at[b],
                         gsem.at[b])

    def wait_scatter(b):
        pltpu.make_async_copy(rows_v.at[b], acc.at[idx_v.at[0]],
                              ssem.at[b]).wait()

    for k0 in range(NBUF - 1):
        @pl.when(k0 < nfull)
        def _():
            gather(k0)

    def step(k, carry):
        b = lax.rem(k, NBUF)

        @pl.when(k + (NBUF - 1) < nfull)
        def _():
            @pl.when(k >= 1)
            def _():
                wait_scatter(lax.rem(k + NBUF - 1, NBUF))
            gather(k + (NBUF - 1))

        pltpu.make_async_copy(emb.at[pl.ds(0, CH)], rows_v.at[b],
                              gsem.at[b]).wait()
        pltpu.async_copy(rows_v.at[b], acc.at[idx_v.at[k]], ssem.at[b],
                         add=True)
        return carry

    lax.fori_loop(0, nfull, step, 0)

    def drain(j, carry):
        wait_scatter(lax.rem(j, NBUF))
        return carry

    lax.fori_loop(jnp.maximum(nfull - NBUF, 0), nfull, drain, 0)
    plsc.subcore_barrier()
    pltpu.sync_copy(acc.at[pl.ds(s * GPS, GPS)], out.at[c, pl.ds(s * GPS, GPS)])


_sc_segsum = functools.partial(
    pl.kernel,
    out_type=jax.ShapeDtypeStruct((NC, G, D), jnp.float32),
    mesh=plsc.VectorSubcoreMesh(core_axis_name="c", subcore_axis_name="s"),
    name="sc_segment_sum",
    scratch_types=[
        pltpu.VMEM((NBUF, CH, D), jnp.float32),
        pltpu.VMEM((MAXCH, CH), jnp.int32),
        pltpu.VMEM_SHARED((G, D), jnp.float32),
        pltpu.SemaphoreType.DMA((NBUF,)),
        pltpu.SemaphoreType.DMA((NBUF,)),
    ],
)(_sc_body)


def _mlp_body(p_ref, w1_ref, b1_ref, w2_ref, b2_ref, o_ref):
    g = p_ref[0] + p_ref[1]
    h = lax.dot_general(g, w1_ref[...], (((1,), (1,)), ((), ())),
                        preferred_element_type=jnp.float32)
    h = jnp.maximum(h + b1_ref[...], 0.0)
    o_ref[...] = lax.dot_general(h, w2_ref[...], (((1,), (1,)), ((), ())),
                                 preferred_element_type=jnp.float32) + b2_ref[...]


_tc_mlp = pl.pallas_call(
    _mlp_body,
    out_shape=jax.ShapeDtypeStruct((G, D), jnp.float32),
)


def kernel(node_embeddings, batch_indices, W1, b1, W2, b2):
    idx = batch_indices.astype(jnp.int32)
    idx3 = jnp.pad(idx, (0, NW * MAXCH * CH - N)).reshape(NW, MAXCH, CH)
    zeros = jnp.zeros((CH, D), jnp.float32)
    partials = _sc_segsum(node_embeddings, idx3, zeros)
    return _tc_mlp(partials, W1, b1.reshape(1, D), W2, b2.reshape(1, D))
